# baseline (speedup 1.0000x reference)
import jax
import jax.numpy as jnp
from jax import lax
from jax.experimental import pallas as pl
from jax.experimental.pallas import tpu as pltpu
from jax.experimental.pallas import tpu_sc as plsc

B = 16384
L = 200
E = 16
V = 1000000

NC, NS = 2, 16
NW = NC * NS
TI = L // 8
TJ = B // 128
NT = TI * TJ
TILE = 1024
TPAD = TILE + 1
IDX_ROWS = 8
TPW = NT // NW

_mesh = plsc.VectorSubcoreMesh(
    core_axis_name="c", subcore_axis_name="s", num_cores=NC, num_subcores=NS
)


def _body(x_hbm, table_hbm, out_hbm,
          idx_s, rows_s, outT_s, isem_s, gsem_s, osem_s):
    wid = lax.axis_index("s") * NC + lax.axis_index("c")
    eoff = lax.iota(jnp.int32, 16)
    t_base = wid * TPW

    def fire_idx(b, t):
        pltpu.async_copy(
            x_hbm.at[pl.ds((t_base + t) * IDX_ROWS, IDX_ROWS)],
            idx_s[b], isem_s[b],
        )

    def fire_gat(b, t):
        del t
        pltpu.make_async_copy(x_hbm.at[pl.ds(0, IDX_ROWS)], idx_s[b],
                              isem_s[b]).wait()
        for j in range(IDX_ROWS):
            pltpu.async_copy(
                table_hbm.at[idx_s[b].at[j]],
                rows_s[b].at[pl.ds(j * 128, 128)],
                gsem_s[b],
            )

    def drain_gathers(b):
        pltpu.make_async_copy(table_hbm.at[pl.ds(0, TILE)], rows_s[b],
                              gsem_s[b]).wait()

    def out_slice(t):
        return out_hbm.at[:, pl.ds((t_base + t) * TILE, TILE)]

    def drain_out(ob):
        pltpu.make_async_copy(outT_s[ob].at[:, pl.ds(0, TILE)], out_slice(0),
                              osem_s[ob]).wait()

    def scat(b, ob):
        rows_v = rows_s[b]
        outT = outT_s[ob]

        @plsc.parallel_loop(0, TILE // 16, unroll=2)
        def _(k16):
            base = k16 * 16
            for kk in range(16):
                k = base + kk
                vec = rows_v[k, :]
                plsc.store_scatter(outT, [eoff, jnp.broadcast_to(k, (16,))], vec)

    for b in range(4):
        fire_idx(b, b)
    fire_gat(0, 0)
    fire_gat(1, 1)
    fire_gat(2, 2)

    def step(q, carry):
        for b in range(4):
            t = 4 * q + b
            ob = b % 2
            drain_gathers(b)

            @pl.when(t + 3 < TPW)
            def _():
                fire_gat((b + 3) % 4, t + 3)

            @pl.when(t > 1)
            def _():
                drain_out(ob)

            scat(b, ob)
            pltpu.async_copy(outT_s[ob].at[:, pl.ds(0, TILE)], out_slice(t),
                             osem_s[ob])

            @pl.when(t + 4 < TPW)
            def _():
                fire_idx(b, t + 4)
        return carry

    lax.fori_loop(0, TPW // 4, step, 0)
    drain_out(0)
    drain_out(1)


_gather_transpose = pl.kernel(
    _body,
    out_type=jax.ShapeDtypeStruct((E, NT * TILE), jnp.float32),
    mesh=_mesh,
    scratch_types=[
        [pltpu.VMEM((IDX_ROWS, 128), jnp.int32) for _ in range(4)],
        [pltpu.VMEM((TILE, E), jnp.float32) for _ in range(4)],
        [pltpu.VMEM((E, TPAD), jnp.float32) for _ in range(2)],
        [pltpu.SemaphoreType.DMA for _ in range(4)],
        [pltpu.SemaphoreType.DMA for _ in range(4)],
        [pltpu.SemaphoreType.DMA for _ in range(2)],
    ],
    compiler_params=pltpu.CompilerParams(
        needs_layout_passes=False, use_tc_tiling_on_sc=False
    ),
)


@jax.jit
def kernel(x, table):
    x_pre = (
        x.transpose(1, 0)
        .reshape(TI, 8, TJ, 128)
        .transpose(0, 2, 1, 3)
        .reshape(NT * IDX_ROWS, 128)
    )
    out2d = _gather_transpose(x_pre, table)
    out = (
        out2d.reshape(E, TI, TJ, 8, 128)
        .transpose(2, 4, 0, 1, 3)
        .reshape(B, E, L)
    )
    return out
